# baseline (device time: 174832 ns/iter reference)
import jax
import jax.numpy as jnp
from jax import lax
from jax.experimental import pallas as pl
from jax.experimental.pallas import tpu as pltpu

N_DEV = 4


def kernel(A, B):
    M, K = A.shape
    _, N = B.shape

    def body(a_ref, b_ref, out_ref, comm_ref, send_sems, recv_sems):
        my_pos = lax.axis_index("i")
        left = (my_pos - 1) % N_DEV
        right = (my_pos + 1) % N_DEV

        barrier_sem = pltpu.get_barrier_semaphore()
        for nbr in [left, right]:
            pl.semaphore_signal(
                barrier_sem, inc=1,
                device_id=(nbr,), device_id_type=pl.DeviceIdType.MESH,
            )
        pl.semaphore_wait(barrier_sem, 2)

        a = a_ref[:, :].astype(jnp.bfloat16)
        b = b_ref[:, :].astype(jnp.bfloat16)
        partial = jnp.dot(a, b, preferred_element_type=jnp.float32)
        out_ref[:, :] = partial
        comm_ref[0, :, :] = partial.astype(jnp.bfloat16)

        for h in range(N_DEV - 1):
            send_slot = h % 2
            recv_slot = (h + 1) % 2
            rdma = pltpu.make_async_remote_copy(
                src_ref=comm_ref.at[send_slot],
                dst_ref=comm_ref.at[recv_slot],
                send_sem=send_sems.at[send_slot],
                recv_sem=recv_sems.at[recv_slot],
                device_id=(right,),
                device_id_type=pl.DeviceIdType.MESH,
            )
            rdma.start()
            rdma.wait()
            out_ref[:, :] += comm_ref[recv_slot, :, :].astype(jnp.float32)

    return pl.pallas_call(
        body,
        out_shape=jax.ShapeDtypeStruct((M, N), jnp.float32),
        in_specs=[
            pl.BlockSpec(memory_space=pltpu.VMEM),
            pl.BlockSpec(memory_space=pltpu.VMEM),
        ],
        out_specs=pl.BlockSpec(memory_space=pltpu.VMEM),
        scratch_shapes=[
            pltpu.VMEM((2, M, N), jnp.bfloat16),
            pltpu.SemaphoreType.DMA((2,)),
            pltpu.SemaphoreType.DMA((2,)),
        ],
        compiler_params=pltpu.CompilerParams(collective_id=0),
    )(A, B)


# device time: 62079 ns/iter; 2.8163x vs baseline; 2.8163x over previous
import jax
import jax.numpy as jnp
from jax import lax
from jax.experimental import pallas as pl
from jax.experimental.pallas import tpu as pltpu

N_DEV = 4


def kernel(A, B):
    M, K = A.shape
    _, N = B.shape
    H = M // 2
    Q = M // 4
    CH = N // 2

    def body(a_ref, b_ref, out_ref,
             sendA, sendB,
             rA1, rB1, rA2, rB2, rA3, rB3, rA4, rB4,
             send_sems, recv_sems):
        p = lax.axis_index("i")
        pA = p ^ 1
        pB = p ^ 3

        kA = (p & 1) ^ ((p >> 1) & 1)
        kB = (p >> 1) & 1
        kAr = (p >> 1) & 1
        kBr = p & 1

        barrier_sem = pltpu.get_barrier_semaphore()
        for nbr in [pA, pB]:
            pl.semaphore_signal(
                barrier_sem, inc=1,
                device_id=(nbr,), device_id_type=pl.DeviceIdType.MESH,
            )
        pl.semaphore_wait(barrier_sem, 2)

        a = a_ref[:, :].astype(jnp.bfloat16)
        b = b_ref[:, :].astype(jnp.bfloat16)
        out_ref[:, :] = jnp.dot(a, b, preferred_element_type=jnp.float32)

        def exchange(sem_idx, src_ref, dst_ref, partner):
            return pltpu.make_async_remote_copy(
                src_ref=src_ref,
                dst_ref=dst_ref,
                send_sem=send_sems.at[sem_idx],
                recv_sem=recv_sems.at[sem_idx],
                device_id=(partner,),
                device_id_type=pl.DeviceIdType.MESH,
            )

        bf16 = jnp.bfloat16
        f32 = jnp.float32

        sendA[:, :] = out_ref[pl.ds((1 - kA) * H, H), pl.ds(0, CH)].astype(bf16)
        sendB[:, :] = out_ref[pl.ds((1 - kAr) * H, H), pl.ds(CH, CH)].astype(bf16)
        r1a = exchange(0, sendA, rA1, pA)
        r1b = exchange(1, sendB, rB1, pB)
        r1a.start()
        r1b.start()
        r1a.wait()
        r1b.wait()
        out_ref[pl.ds(kA * H, H), pl.ds(0, CH)] += rA1[:, :].astype(f32)
        out_ref[pl.ds(kAr * H, H), pl.ds(CH, CH)] += rB1[:, :].astype(f32)

        sendB[pl.ds(0, Q), :] = out_ref[
            pl.ds(kA * H + (1 - kB) * Q, Q), pl.ds(0, CH)
        ].astype(bf16)
        sendA[pl.ds(0, Q), :] = out_ref[
            pl.ds(kAr * H + (1 - kBr) * Q, Q), pl.ds(CH, CH)
        ].astype(bf16)
        r2b = exchange(2, sendB.at[pl.ds(0, Q)], rB2, pB)
        r2a = exchange(3, sendA.at[pl.ds(0, Q)], rA2, pA)
        r2b.start()
        r2a.start()
        r2b.wait()
        r2a.wait()
        out_ref[pl.ds(kA * H + kB * Q, Q), pl.ds(0, CH)] += rB2[:, :].astype(f32)
        out_ref[pl.ds(kAr * H + kBr * Q, Q), pl.ds(CH, CH)] += rA2[:, :].astype(f32)

        sendB[pl.ds(0, Q), :] = out_ref[
            pl.ds(kA * H + kB * Q, Q), pl.ds(0, CH)
        ].astype(bf16)
        sendA[pl.ds(0, Q), :] = out_ref[
            pl.ds(kAr * H + kBr * Q, Q), pl.ds(CH, CH)
        ].astype(bf16)
        r3b = exchange(4, sendB.at[pl.ds(0, Q)], rB3, pB)
        r3a = exchange(5, sendA.at[pl.ds(0, Q)], rA3, pA)
        r3b.start()
        r3a.start()
        r3b.wait()
        r3a.wait()
        out_ref[pl.ds(kA * H + (1 - kB) * Q, Q), pl.ds(0, CH)] = (
            rB3[:, :].astype(f32)
        )
        out_ref[pl.ds(kAr * H + (1 - kBr) * Q, Q), pl.ds(CH, CH)] = (
            rA3[:, :].astype(f32)
        )

        sendA[:, :] = out_ref[pl.ds(kA * H, H), pl.ds(0, CH)].astype(bf16)
        sendB[:, :] = out_ref[pl.ds(kAr * H, H), pl.ds(CH, CH)].astype(bf16)
        r4a = exchange(6, sendA, rA4, pA)
        r4b = exchange(7, sendB, rB4, pB)
        r4a.start()
        r4b.start()
        r4a.wait()
        r4b.wait()
        out_ref[pl.ds((1 - kA) * H, H), pl.ds(0, CH)] = rA4[:, :].astype(f32)
        out_ref[pl.ds((1 - kAr) * H, H), pl.ds(CH, CH)] = rB4[:, :].astype(f32)

    return pl.pallas_call(
        body,
        out_shape=jax.ShapeDtypeStruct((M, N), jnp.float32),
        in_specs=[
            pl.BlockSpec(memory_space=pltpu.VMEM),
            pl.BlockSpec(memory_space=pltpu.VMEM),
        ],
        out_specs=pl.BlockSpec(memory_space=pltpu.VMEM),
        scratch_shapes=[
            pltpu.VMEM((H, CH), jnp.bfloat16),
            pltpu.VMEM((H, CH), jnp.bfloat16),
            pltpu.VMEM((H, CH), jnp.bfloat16),
            pltpu.VMEM((H, CH), jnp.bfloat16),
            pltpu.VMEM((Q, CH), jnp.bfloat16),
            pltpu.VMEM((Q, CH), jnp.bfloat16),
            pltpu.VMEM((Q, CH), jnp.bfloat16),
            pltpu.VMEM((Q, CH), jnp.bfloat16),
            pltpu.VMEM((H, CH), jnp.bfloat16),
            pltpu.VMEM((H, CH), jnp.bfloat16),
            pltpu.SemaphoreType.DMA((8,)),
            pltpu.SemaphoreType.DMA((8,)),
        ],
        compiler_params=pltpu.CompilerParams(collective_id=0),
    )(A, B)


# device time: 61099 ns/iter; 2.8615x vs baseline; 1.0160x over previous
import jax
import jax.numpy as jnp
from jax import lax
from jax.experimental import pallas as pl
from jax.experimental.pallas import tpu as pltpu

N_DEV = 4


def kernel(A, B):
    M, K = A.shape
    _, N = B.shape
    H = M // 2
    Q = M // 4
    CH = N // 2

    def body(a_ref, b_ref, out_ref,
             sendA, sendB,
             rA1, rB1, rA2, rB2, rA3, rB3, rA4, rB4,
             send_sems, recv_sems):
        p = lax.axis_index("i")
        pA = p ^ 1
        pB = p ^ 3

        kA = (p & 1) ^ ((p >> 1) & 1)
        kB = (p >> 1) & 1
        kAr = (p >> 1) & 1
        kBr = p & 1

        barrier_sem = pltpu.get_barrier_semaphore()
        for nbr in [pA, pB]:
            pl.semaphore_signal(
                barrier_sem, inc=1,
                device_id=(nbr,), device_id_type=pl.DeviceIdType.MESH,
            )
        pl.semaphore_wait(barrier_sem, 2)

        bf16 = jnp.bfloat16
        f32 = jnp.float32

        def exchange(sem_idx, src_ref, dst_ref, partner):
            return pltpu.make_async_remote_copy(
                src_ref=src_ref,
                dst_ref=dst_ref,
                send_sem=send_sems.at[sem_idx],
                recv_sem=recv_sems.at[sem_idx],
                device_id=(partner,),
                device_id_type=pl.DeviceIdType.MESH,
            )

        def dot_block(row0, col0):
            a_blk = a_ref[pl.ds(row0, H), :].astype(bf16)
            b_blk = b_ref[:, pl.ds(col0, CH)].astype(bf16)
            return jnp.dot(a_blk, b_blk, preferred_element_type=f32)

        sL = dot_block((1 - kA) * H, 0)
        out_ref[pl.ds((1 - kA) * H, H), pl.ds(0, CH)] = sL
        sendA[:, :] = sL.astype(bf16)
        r1a = exchange(0, sendA, rA1, pA)
        r1a.start()

        sR = dot_block((1 - kAr) * H, CH)
        out_ref[pl.ds((1 - kAr) * H, H), pl.ds(CH, CH)] = sR
        sendB[:, :] = sR.astype(bf16)
        r1b = exchange(1, sendB, rB1, pB)
        r1b.start()

        kL = dot_block(kA * H, 0)
        kR = dot_block(kAr * H, CH)

        r1a.wait()
        s1 = kL + rA1[:, :].astype(f32)
        out_ref[pl.ds(kA * H, H), pl.ds(0, CH)] = s1
        sendB[pl.ds(0, Q), :] = out_ref[
            pl.ds(kA * H + (1 - kB) * Q, Q), pl.ds(0, CH)
        ].astype(bf16)
        r2b = exchange(2, sendB.at[pl.ds(0, Q)], rB2, pB)
        r2b.start()

        r1b.wait()
        s1r = kR + rB1[:, :].astype(f32)
        out_ref[pl.ds(kAr * H, H), pl.ds(CH, CH)] = s1r
        sendA[pl.ds(0, Q), :] = out_ref[
            pl.ds(kAr * H + (1 - kBr) * Q, Q), pl.ds(CH, CH)
        ].astype(bf16)
        r2a = exchange(3, sendA.at[pl.ds(0, Q)], rA2, pA)
        r2a.start()

        r2b.wait()
        s2 = (
            out_ref[pl.ds(kA * H + kB * Q, Q), pl.ds(0, CH)]
            + rB2[:, :].astype(f32)
        )
        out_ref[pl.ds(kA * H + kB * Q, Q), pl.ds(0, CH)] = s2
        sendB[pl.ds(0, Q), :] = s2.astype(bf16)
        r3b = exchange(4, sendB.at[pl.ds(0, Q)], rB3, pB)
        r3b.start()

        r2a.wait()
        s2r = (
            out_ref[pl.ds(kAr * H + kBr * Q, Q), pl.ds(CH, CH)]
            + rA2[:, :].astype(f32)
        )
        out_ref[pl.ds(kAr * H + kBr * Q, Q), pl.ds(CH, CH)] = s2r
        sendA[pl.ds(0, Q), :] = s2r.astype(bf16)
        r3a = exchange(5, sendA.at[pl.ds(0, Q)], rA3, pA)
        r3a.start()

        r3b.wait()
        out_ref[pl.ds(kA * H + (1 - kB) * Q, Q), pl.ds(0, CH)] = (
            rB3[:, :].astype(f32)
        )
        r4a1 = exchange(6, sendB.at[pl.ds(0, Q)], rA4.at[pl.ds(kB * Q, Q)], pA)
        r4a2 = exchange(7, rB3, rA4.at[pl.ds((1 - kB) * Q, Q)], pA)
        r4a1.start()
        r4a2.start()

        r3a.wait()
        out_ref[pl.ds(kAr * H + (1 - kBr) * Q, Q), pl.ds(CH, CH)] = (
            rA3[:, :].astype(f32)
        )
        r4b1 = exchange(8, sendA.at[pl.ds(0, Q)], rB4.at[pl.ds(kBr * Q, Q)], pB)
        r4b2 = exchange(9, rA3, rB4.at[pl.ds((1 - kBr) * Q, Q)], pB)
        r4b1.start()
        r4b2.start()

        r4a1.wait()
        r4a2.wait()
        out_ref[pl.ds((1 - kA) * H, H), pl.ds(0, CH)] = rA4[:, :].astype(f32)

        r4b1.wait()
        r4b2.wait()
        out_ref[pl.ds((1 - kAr) * H, H), pl.ds(CH, CH)] = rB4[:, :].astype(f32)

    return pl.pallas_call(
        body,
        out_shape=jax.ShapeDtypeStruct((M, N), jnp.float32),
        in_specs=[
            pl.BlockSpec(memory_space=pltpu.VMEM),
            pl.BlockSpec(memory_space=pltpu.VMEM),
        ],
        out_specs=pl.BlockSpec(memory_space=pltpu.VMEM),
        scratch_shapes=[
            pltpu.VMEM((H, CH), jnp.bfloat16),
            pltpu.VMEM((H, CH), jnp.bfloat16),
            pltpu.VMEM((H, CH), jnp.bfloat16),
            pltpu.VMEM((H, CH), jnp.bfloat16),
            pltpu.VMEM((Q, CH), jnp.bfloat16),
            pltpu.VMEM((Q, CH), jnp.bfloat16),
            pltpu.VMEM((Q, CH), jnp.bfloat16),
            pltpu.VMEM((Q, CH), jnp.bfloat16),
            pltpu.VMEM((H, CH), jnp.bfloat16),
            pltpu.VMEM((H, CH), jnp.bfloat16),
            pltpu.SemaphoreType.DMA((10,)),
            pltpu.SemaphoreType.DMA((10,)),
        ],
        compiler_params=pltpu.CompilerParams(collective_id=0),
    )(A, B)
